# Initial kernel scaffold; baseline (speedup 1.0000x reference)
import jax
import jax.numpy as jnp
from jax.experimental import pallas as pl


def kernel(x, edge_index, edge_colour, Wc1, Wl1, b1, Wc2, Wl2, b2):
    raise NotImplementedError("write your pallas kernel here")


# speedup vs baseline: 3.6408x; 3.6408x over previous
import functools

import jax
import jax.numpy as jnp
from jax import lax
from jax.experimental import pallas as pl
from jax.experimental.pallas import tpu as pltpu
from jax.experimental.pallas import tpu_sc as plsc

N = 10000
E = 320000
D = 128
H = 256
C = 4

NTILES = 16
E_TILE = E // NTILES
CHUNK = 128
E_PAD = ((E_TILE + CHUNK - 1) // CHUNK) * CHUNK
NCHUNK = E_PAD // CHUNK
TROWS = 10112
DUMP = N
ZSTRIPE = TROWS // NTILES
DSTRIPE = 624
DREM = N - NTILES * DSTRIPE


def _make_sc_agg(num_srcs):
    mesh = plsc.VectorSubcoreMesh(core_axis_name="c", subcore_axis_name="s")

    @functools.partial(
        pl.kernel,
        mesh=mesh,
        out_type=jax.ShapeDtypeStruct((C, num_srcs, N, D), jnp.float32),
        scratch_types=[
            pltpu.VMEM((E_PAD,), jnp.int32),
            pltpu.VMEM((1, CHUNK), jnp.int32),
            pltpu.VMEM((1, CHUNK), jnp.int32),
            pltpu.VMEM((CHUNK, D), jnp.float32),
            pltpu.VMEM_SHARED((TROWS, D), jnp.float32),
            pltpu.SemaphoreType.DMA,
        ],
    )
    def agg_kernel(*refs):
        feats = refs[:num_srcs]
        pk_hbm, zeros_hbm, out_hbm = refs[num_srcs:num_srcs + 3]
        pk_v, gidx_v, sidx_v, rows_v, table, sem = refs[num_srcs + 3:]

        cid = lax.axis_index("c")
        sid = lax.axis_index("s")

        eb = sid * E_TILE
        pltpu.sync_copy(pk_hbm.at[pl.ds(eb, E_TILE)], pk_v.at[pl.ds(0, E_TILE)])
        for k in range((E_PAD - E_TILE) // 16):
            pk_v[pl.ds(E_TILE + k * 16, 16)] = jnp.full((16,), DUMP << 14,
                                                        jnp.int32)

        for f in range(num_srcs):
            feat = feats[f]
            for r in range(2):
                q = cid * 2 + r

                zb = sid * ZSTRIPE
                pltpu.sync_copy(zeros_hbm.at[pl.ds(zb, ZSTRIPE)],
                                table.at[pl.ds(zb, ZSTRIPE)])
                plsc.subcore_barrier()

                def chunk_body(i, carry):
                    off = i * CHUNK
                    for j in range(CHUNK // 16):
                        p16 = pk_v[pl.ds(off + j * 16, 16)]
                        s16 = p16 & 0x3FFF
                        d16 = (p16 >> 14) & 0x3FFF
                        c16 = p16 >> 28
                        gidx_v[0, pl.ds(j * 16, 16)] = s16
                        sidx_v[0, pl.ds(j * 16, 16)] = jnp.where(
                            c16 == q, d16, DUMP)
                    pltpu.async_copy(feat.at[gidx_v.at[0]],
                                     rows_v, sem).wait()
                    pltpu.sync_copy(rows_v, table.at[sidx_v.at[0]], add=True)
                    return carry

                lax.fori_loop(0, NCHUNK, chunk_body, 0)
                plsc.subcore_barrier()

                db = sid * DSTRIPE
                for qq in range(C):
                    @pl.when(q == qq)
                    def _():
                        pltpu.sync_copy(
                            table.at[pl.ds(db, DSTRIPE)],
                            out_hbm.at[qq, f, pl.ds(db, DSTRIPE)])

                    @pl.when(jnp.logical_and(q == qq, sid == 0))
                    def _():
                        pltpu.sync_copy(
                            table.at[pl.ds(NTILES * DSTRIPE, DREM)],
                            out_hbm.at[qq, f, pl.ds(NTILES * DSTRIPE, DREM)])
                plsc.subcore_barrier()

    return agg_kernel


_sc_agg1 = _make_sc_agg(1)
_sc_agg2 = _make_sc_agg(2)


BN = 1000


def _tc1_body(x_ref, agg_ref, wl_ref, wc_ref, b_ref, ha_ref, hb_ref):
    acc = jnp.dot(x_ref[...], wl_ref[...], preferred_element_type=jnp.float32)
    for c in range(C):
        acc += jnp.dot(agg_ref[c, 0], wc_ref[c],
                       preferred_element_type=jnp.float32)
    acc += b_ref[...]
    h = jnp.maximum(acc, 0.0)
    ha_ref[...] = h[:, :D]
    hb_ref[...] = h[:, D:]


def _tc_layer1(x, agg1, wl1t, wc1t, b1):
    return pl.pallas_call(
        _tc1_body,
        grid=(N // BN,),
        in_specs=[
            pl.BlockSpec((BN, D), lambda i: (i, 0)),
            pl.BlockSpec((C, 1, BN, D), lambda i: (0, 0, i, 0)),
            pl.BlockSpec((D, H), lambda i: (0, 0)),
            pl.BlockSpec((C, D, H), lambda i: (0, 0, 0)),
            pl.BlockSpec((1, H), lambda i: (0, 0)),
        ],
        out_specs=[
            pl.BlockSpec((BN, D), lambda i: (i, 0)),
            pl.BlockSpec((BN, D), lambda i: (i, 0)),
        ],
        out_shape=[
            jax.ShapeDtypeStruct((N, D), jnp.float32),
            jax.ShapeDtypeStruct((N, D), jnp.float32),
        ],
    )(x, agg1, wl1t, wc1t, b1)


def _tc2_body(ha_ref, hb_ref, agg_ref, wl_ref, wc_ref, b_ref, o_ref):
    acc = jnp.dot(ha_ref[...], wl_ref[:D], preferred_element_type=jnp.float32)
    acc += jnp.dot(hb_ref[...], wl_ref[D:], preferred_element_type=jnp.float32)
    for c in range(C):
        for f in range(2):
            acc += jnp.dot(agg_ref[c, f], wc_ref[c, f * D:(f + 1) * D],
                           preferred_element_type=jnp.float32)
    acc += b_ref[...]
    o_ref[...] = jax.nn.sigmoid(acc - 10.0)


def _tc_layer2(ha, hb, agg2, wl2t, wc2t, b2):
    return pl.pallas_call(
        _tc2_body,
        grid=(N // BN,),
        in_specs=[
            pl.BlockSpec((BN, D), lambda i: (i, 0)),
            pl.BlockSpec((BN, D), lambda i: (i, 0)),
            pl.BlockSpec((C, 2, BN, D), lambda i: (0, 0, i, 0)),
            pl.BlockSpec((H, D), lambda i: (0, 0)),
            pl.BlockSpec((C, H, D), lambda i: (0, 0, 0)),
            pl.BlockSpec((1, D), lambda i: (0, 0)),
        ],
        out_specs=pl.BlockSpec((BN, D), lambda i: (i, 0)),
        out_shape=jax.ShapeDtypeStruct((N, D), jnp.float32),
    )(ha, hb, agg2, wl2t, wc2t, b2)


def kernel(x, edge_index, edge_colour, Wc1, Wl1, b1, Wc2, Wl2, b2):
    packed = (edge_index[0] | (edge_index[1] << 14)
              | (edge_colour << 28)).astype(jnp.int32)
    zeros = jnp.zeros((TROWS, D), jnp.float32)

    agg1 = _sc_agg1(x, packed, zeros)
    ha, hb = _tc_layer1(x, agg1, Wl1.T,
                        jnp.transpose(Wc1, (0, 2, 1)), b1.reshape(1, H))
    agg2 = _sc_agg2(ha, hb, packed, zeros)
    out = _tc_layer2(ha, hb, agg2, Wl2.T,
                     jnp.transpose(Wc2, (0, 2, 1)), b2.reshape(1, D))
    return out


# speedup vs baseline: 4.0969x; 1.1253x over previous
import functools

import jax
import jax.numpy as jnp
from jax import lax
from jax.experimental import pallas as pl
from jax.experimental.pallas import tpu as pltpu
from jax.experimental.pallas import tpu_sc as plsc

N = 10000
E = 320000
D = 128
H = 256
C = 4

NTILES = 16
E_TILE = E // NTILES
CHUNK = 96
NGRP = CHUNK // 16
NCHUNK = (E_TILE + CHUNK - 1) // CHUNK
E_PAD = NCHUNK * CHUNK
NPAIR = NCHUNK // 2
TROWS = 10112
DUMP = N
ZSTRIPE = TROWS // NTILES
DSTRIPE = 624
DREM = N - NTILES * DSTRIPE
MASK14 = 0x3FFF


def _make_sc_agg(num_srcs):
    mesh = plsc.VectorSubcoreMesh(core_axis_name="c", subcore_axis_name="s")

    @functools.partial(
        pl.kernel,
        mesh=mesh,
        out_type=jax.ShapeDtypeStruct((C, num_srcs, N, D), jnp.float32),
        scratch_types=[
            pltpu.VMEM((E_PAD,), jnp.int32),
            pltpu.VMEM((1, CHUNK), jnp.int32),
            pltpu.VMEM((1, CHUNK), jnp.int32),
            pltpu.VMEM((CHUNK, D), jnp.float32),
            pltpu.VMEM((1, CHUNK), jnp.int32),
            pltpu.VMEM((1, CHUNK), jnp.int32),
            pltpu.VMEM((CHUNK, D), jnp.float32),
            pltpu.VMEM_SHARED((TROWS, D), jnp.float32),
            pltpu.SemaphoreType.DMA,
            pltpu.SemaphoreType.DMA,
            pltpu.SemaphoreType.DMA,
        ],
    )
    def agg_kernel(*refs):
        feats = refs[:num_srcs]
        pk_hbm, zeros_hbm, out_hbm = refs[num_srcs:num_srcs + 3]
        (pk_v, gidx_a, sidx_a, rows_a, gidx_b, sidx_b, rows_b,
         table, sem_a, sem_b, sem_s) = refs[num_srcs + 3:]

        cid = lax.axis_index("c")
        sid = lax.axis_index("s")

        eb = sid * E_TILE
        pltpu.sync_copy(pk_hbm.at[pl.ds(eb, E_TILE)], pk_v.at[pl.ds(0, E_TILE)])
        for k in range((E_PAD - E_TILE) // 16):
            pk_v[pl.ds(E_TILE + k * 16, 16)] = jnp.full((16,), DUMP << 14,
                                                        jnp.int32)

        for r in range(2):
            q = cid * 2 + r

            def unpack(off, gidx, sidx):
                for j in range(NGRP):
                    p16 = pk_v[pl.ds(off + j * 16, 16)]
                    gidx[0, pl.ds(j * 16, 16)] = p16 & MASK14
                    sidx[0, pl.ds(j * 16, 16)] = jnp.where(
                        (p16 >> 28) == q, (p16 >> 14) & MASK14, DUMP)

            for f in range(num_srcs):
                feat = feats[f]

                zb = sid * ZSTRIPE
                pltpu.sync_copy(zeros_hbm.at[pl.ds(zb, ZSTRIPE)],
                                table.at[pl.ds(zb, ZSTRIPE)])
                plsc.subcore_barrier()

                def pair_body(k, carry):
                    off_a = (2 * k) * CHUNK
                    off_b = off_a + CHUNK
                    unpack(off_a, gidx_a, sidx_a)
                    ga = pltpu.async_copy(feat.at[gidx_a.at[0]], rows_a,
                                          sem_a)
                    unpack(off_b, gidx_b, sidx_b)
                    gb = pltpu.async_copy(feat.at[gidx_b.at[0]], rows_b,
                                          sem_b)
                    ga.wait()
                    pltpu.async_copy(rows_a, table.at[sidx_a.at[0]], sem_s,
                                     add=True).wait()
                    gb.wait()
                    pltpu.async_copy(rows_b, table.at[sidx_b.at[0]], sem_s,
                                     add=True).wait()
                    return carry

                lax.fori_loop(0, NPAIR, pair_body, 0)
                unpack((NCHUNK - 1) * CHUNK, gidx_a, sidx_a)
                pltpu.async_copy(feat.at[gidx_a.at[0]], rows_a, sem_a).wait()
                pltpu.async_copy(rows_a, table.at[sidx_a.at[0]], sem_s,
                                 add=True).wait()
                plsc.subcore_barrier()

                db = sid * DSTRIPE
                for qq in range(C):
                    @pl.when(q == qq)
                    def _():
                        pltpu.sync_copy(
                            table.at[pl.ds(db, DSTRIPE)],
                            out_hbm.at[qq, f, pl.ds(db, DSTRIPE)])

                    @pl.when(jnp.logical_and(q == qq, sid == 0))
                    def _():
                        pltpu.sync_copy(
                            table.at[pl.ds(NTILES * DSTRIPE, DREM)],
                            out_hbm.at[qq, f, pl.ds(NTILES * DSTRIPE, DREM)])
                plsc.subcore_barrier()

    return agg_kernel


_sc_agg1 = _make_sc_agg(1)
_sc_agg2 = _make_sc_agg(2)


BN = 1000


def _tc1_body(x_ref, agg_ref, wl_ref, wc_ref, b_ref, ha_ref, hb_ref):
    acc = jnp.dot(x_ref[...], wl_ref[...], preferred_element_type=jnp.float32)
    for c in range(C):
        acc += jnp.dot(agg_ref[c, 0], wc_ref[c],
                       preferred_element_type=jnp.float32)
    acc += b_ref[...]
    h = jnp.maximum(acc, 0.0)
    ha_ref[...] = h[:, :D]
    hb_ref[...] = h[:, D:]


def _tc_layer1(x, agg1, wl1t, wc1t, b1):
    return pl.pallas_call(
        _tc1_body,
        grid=(N // BN,),
        in_specs=[
            pl.BlockSpec((BN, D), lambda i: (i, 0)),
            pl.BlockSpec((C, 1, BN, D), lambda i: (0, 0, i, 0)),
            pl.BlockSpec((D, H), lambda i: (0, 0)),
            pl.BlockSpec((C, D, H), lambda i: (0, 0, 0)),
            pl.BlockSpec((1, H), lambda i: (0, 0)),
        ],
        out_specs=[
            pl.BlockSpec((BN, D), lambda i: (i, 0)),
            pl.BlockSpec((BN, D), lambda i: (i, 0)),
        ],
        out_shape=[
            jax.ShapeDtypeStruct((N, D), jnp.float32),
            jax.ShapeDtypeStruct((N, D), jnp.float32),
        ],
    )(x, agg1, wl1t, wc1t, b1)


def _tc2_body(ha_ref, hb_ref, agg_ref, wl_ref, wc_ref, b_ref, o_ref):
    acc = jnp.dot(ha_ref[...], wl_ref[:D], preferred_element_type=jnp.float32)
    acc += jnp.dot(hb_ref[...], wl_ref[D:], preferred_element_type=jnp.float32)
    for c in range(C):
        for f in range(2):
            acc += jnp.dot(agg_ref[c, f], wc_ref[c, f * D:(f + 1) * D],
                           preferred_element_type=jnp.float32)
    acc += b_ref[...]
    o_ref[...] = jax.nn.sigmoid(acc - 10.0)


def _tc_layer2(ha, hb, agg2, wl2t, wc2t, b2):
    return pl.pallas_call(
        _tc2_body,
        grid=(N // BN,),
        in_specs=[
            pl.BlockSpec((BN, D), lambda i: (i, 0)),
            pl.BlockSpec((BN, D), lambda i: (i, 0)),
            pl.BlockSpec((C, 2, BN, D), lambda i: (0, 0, i, 0)),
            pl.BlockSpec((H, D), lambda i: (0, 0)),
            pl.BlockSpec((C, H, D), lambda i: (0, 0, 0)),
            pl.BlockSpec((1, D), lambda i: (0, 0)),
        ],
        out_specs=pl.BlockSpec((BN, D), lambda i: (i, 0)),
        out_shape=jax.ShapeDtypeStruct((N, D), jnp.float32),
    )(ha, hb, agg2, wl2t, wc2t, b2)


def kernel(x, edge_index, edge_colour, Wc1, Wl1, b1, Wc2, Wl2, b2):
    packed = (edge_index[0] | (edge_index[1] << 14)
              | (edge_colour << 28)).astype(jnp.int32)
    zeros = jnp.zeros((TROWS, D), jnp.float32)

    agg1 = _sc_agg1(x, packed, zeros)
    ha, hb = _tc_layer1(x, agg1, Wl1.T,
                        jnp.transpose(Wc1, (0, 2, 1)), b1.reshape(1, H))
    agg2 = _sc_agg2(ha, hb, packed, zeros)
    out = _tc_layer2(ha, hb, agg2, Wl2.T,
                     jnp.transpose(Wc2, (0, 2, 1)), b2.reshape(1, D))
    return out


# speedup vs baseline: 4.4549x; 1.0874x over previous
import functools

import jax
import jax.numpy as jnp
from jax import lax
from jax.experimental import pallas as pl
from jax.experimental.pallas import tpu as pltpu
from jax.experimental.pallas import tpu_sc as plsc

N = 10000
E = 320000
D = 128
H = 256
C = 4

NTILES = 16
E_TILE = E // NTILES
CHUNK = 96
NGRP = CHUNK // 16
NCHUNK = (E_TILE + CHUNK - 1) // CHUNK
E_PAD = NCHUNK * CHUNK
NPAIR = NCHUNK // 2
TROWS = 10240
DUMP = N
ZSTRIPE = TROWS // NTILES
DSTRIPE = 624
DREM = N - NTILES * DSTRIPE
MASK14 = 0x3FFF


def _make_sc_agg(num_srcs):
    mesh = plsc.VectorSubcoreMesh(core_axis_name="c", subcore_axis_name="s")

    @functools.partial(
        pl.kernel,
        mesh=mesh,
        out_type=jax.ShapeDtypeStruct((C, num_srcs, N, D), jnp.float32),
        scratch_types=[
            pltpu.VMEM((E_PAD,), jnp.int32),
            pltpu.VMEM((1, CHUNK), jnp.int32),
            pltpu.VMEM((1, CHUNK), jnp.int32),
            pltpu.VMEM((CHUNK, D), jnp.float32),
            pltpu.VMEM((1, CHUNK), jnp.int32),
            pltpu.VMEM((1, CHUNK), jnp.int32),
            pltpu.VMEM((CHUNK, D), jnp.float32),
            pltpu.VMEM_SHARED((TROWS, D), jnp.float32),
            pltpu.SemaphoreType.DMA,
            pltpu.SemaphoreType.DMA,
            pltpu.SemaphoreType.DMA,
        ],
    )
    def agg_kernel(*refs):
        feats = refs[:num_srcs]
        pk_hbm, zeros_hbm, out_hbm = refs[num_srcs:num_srcs + 3]
        (pk_v, gidx_a, sidx_a, rows_a, gidx_b, sidx_b, rows_b,
         table, sem_a, sem_b, sem_s) = refs[num_srcs + 3:]

        cid = lax.axis_index("c")
        sid = lax.axis_index("s")

        eb = sid * E_TILE
        pltpu.sync_copy(pk_hbm.at[pl.ds(eb, E_TILE)], pk_v.at[pl.ds(0, E_TILE)])
        for k in range((E_PAD - E_TILE) // 16):
            pk_v[pl.ds(E_TILE + k * 16, 16)] = jnp.full((16,), DUMP << 14,
                                                        jnp.int32)

        for r in range(2):
            q = cid * 2 + r

            def unpack(off, gidx, sidx):
                for j in range(NGRP):
                    p16 = pk_v[pl.ds(off + j * 16, 16)]
                    gidx[0, pl.ds(j * 16, 16)] = p16 & MASK14
                    sidx[0, pl.ds(j * 16, 16)] = jnp.where(
                        (p16 >> 28) == q, (p16 >> 14) & MASK14,
                        DUMP + sid * 8 + (j % 8))

            for f in range(num_srcs):
                feat = feats[f]

                zb = sid * ZSTRIPE
                pltpu.sync_copy(zeros_hbm.at[pl.ds(zb, ZSTRIPE)],
                                table.at[pl.ds(zb, ZSTRIPE)])
                plsc.subcore_barrier()

                def pair_body(k, carry):
                    off_a = (2 * k) * CHUNK
                    off_b = off_a + CHUNK
                    unpack(off_a, gidx_a, sidx_a)
                    ga = pltpu.async_copy(feat.at[gidx_a.at[0]], rows_a,
                                          sem_a)
                    unpack(off_b, gidx_b, sidx_b)
                    gb = pltpu.async_copy(feat.at[gidx_b.at[0]], rows_b,
                                          sem_b)
                    ga.wait()
                    pltpu.async_copy(rows_a, table.at[sidx_a.at[0]], sem_s,
                                     add=True).wait()
                    gb.wait()
                    pltpu.async_copy(rows_b, table.at[sidx_b.at[0]], sem_s,
                                     add=True).wait()
                    return carry

                lax.fori_loop(0, NPAIR, pair_body, 0)
                unpack((NCHUNK - 1) * CHUNK, gidx_a, sidx_a)
                pltpu.async_copy(feat.at[gidx_a.at[0]], rows_a, sem_a).wait()
                pltpu.async_copy(rows_a, table.at[sidx_a.at[0]], sem_s,
                                 add=True).wait()
                plsc.subcore_barrier()

                db = sid * DSTRIPE
                for qq in range(C):
                    @pl.when(q == qq)
                    def _():
                        pltpu.sync_copy(
                            table.at[pl.ds(db, DSTRIPE)],
                            out_hbm.at[qq, f, pl.ds(db, DSTRIPE)])

                    @pl.when(jnp.logical_and(q == qq, sid == 0))
                    def _():
                        pltpu.sync_copy(
                            table.at[pl.ds(NTILES * DSTRIPE, DREM)],
                            out_hbm.at[qq, f, pl.ds(NTILES * DSTRIPE, DREM)])
                plsc.subcore_barrier()

    return agg_kernel


_sc_agg1 = _make_sc_agg(1)
_sc_agg2 = _make_sc_agg(2)


BN = 1000


def _tc1_body(x_ref, agg_ref, wl_ref, wc_ref, b_ref, ha_ref, hb_ref):
    acc = jnp.dot(x_ref[...], wl_ref[...], preferred_element_type=jnp.float32)
    for c in range(C):
        acc += jnp.dot(agg_ref[c, 0], wc_ref[c],
                       preferred_element_type=jnp.float32)
    acc += b_ref[...]
    h = jnp.maximum(acc, 0.0)
    ha_ref[...] = h[:, :D]
    hb_ref[...] = h[:, D:]


def _tc_layer1(x, agg1, wl1t, wc1t, b1):
    return pl.pallas_call(
        _tc1_body,
        grid=(N // BN,),
        in_specs=[
            pl.BlockSpec((BN, D), lambda i: (i, 0)),
            pl.BlockSpec((C, 1, BN, D), lambda i: (0, 0, i, 0)),
            pl.BlockSpec((D, H), lambda i: (0, 0)),
            pl.BlockSpec((C, D, H), lambda i: (0, 0, 0)),
            pl.BlockSpec((1, H), lambda i: (0, 0)),
        ],
        out_specs=[
            pl.BlockSpec((BN, D), lambda i: (i, 0)),
            pl.BlockSpec((BN, D), lambda i: (i, 0)),
        ],
        out_shape=[
            jax.ShapeDtypeStruct((N, D), jnp.float32),
            jax.ShapeDtypeStruct((N, D), jnp.float32),
        ],
    )(x, agg1, wl1t, wc1t, b1)


def _tc2_body(ha_ref, hb_ref, agg_ref, wl_ref, wc_ref, b_ref, o_ref):
    acc = jnp.dot(ha_ref[...], wl_ref[:D], preferred_element_type=jnp.float32)
    acc += jnp.dot(hb_ref[...], wl_ref[D:], preferred_element_type=jnp.float32)
    for c in range(C):
        for f in range(2):
            acc += jnp.dot(agg_ref[c, f], wc_ref[c, f * D:(f + 1) * D],
                           preferred_element_type=jnp.float32)
    acc += b_ref[...]
    o_ref[...] = jax.nn.sigmoid(acc - 10.0)


def _tc_layer2(ha, hb, agg2, wl2t, wc2t, b2):
    return pl.pallas_call(
        _tc2_body,
        grid=(N // BN,),
        in_specs=[
            pl.BlockSpec((BN, D), lambda i: (i, 0)),
            pl.BlockSpec((BN, D), lambda i: (i, 0)),
            pl.BlockSpec((C, 2, BN, D), lambda i: (0, 0, i, 0)),
            pl.BlockSpec((H, D), lambda i: (0, 0)),
            pl.BlockSpec((C, H, D), lambda i: (0, 0, 0)),
            pl.BlockSpec((1, D), lambda i: (0, 0)),
        ],
        out_specs=pl.BlockSpec((BN, D), lambda i: (i, 0)),
        out_shape=jax.ShapeDtypeStruct((N, D), jnp.float32),
    )(ha, hb, agg2, wl2t, wc2t, b2)


def kernel(x, edge_index, edge_colour, Wc1, Wl1, b1, Wc2, Wl2, b2):
    packed = (edge_index[0] | (edge_index[1] << 14)
              | (edge_colour << 28)).astype(jnp.int32)
    zeros = jnp.zeros((TROWS, D), jnp.float32)

    agg1 = _sc_agg1(x, packed, zeros)
    ha, hb = _tc_layer1(x, agg1, Wl1.T,
                        jnp.transpose(Wc1, (0, 2, 1)), b1.reshape(1, H))
    agg2 = _sc_agg2(ha, hb, packed, zeros)
    out = _tc_layer2(ha, hb, agg2, Wl2.T,
                     jnp.transpose(Wc2, (0, 2, 1)), b2.reshape(1, D))
    return out


# speedup vs baseline: 4.5156x; 1.0136x over previous
import functools

import jax
import jax.numpy as jnp
from jax import lax
from jax.experimental import pallas as pl
from jax.experimental.pallas import tpu as pltpu
from jax.experimental.pallas import tpu_sc as plsc

N = 10000
E = 320000
D = 128
H = 256
C = 4

NTILES = 16
E_TILE = E // NTILES
CHUNK = 96
NGRP = CHUNK // 16
NCHUNK = (E_TILE + CHUNK - 1) // CHUNK
E_PAD = NCHUNK * CHUNK
NPAIR = NCHUNK // 2
TROWS = 10240
DUMP = N
ZSTRIPE = TROWS // NTILES
DSTRIPE = 624
DREM = N - NTILES * DSTRIPE
MASK14 = 0x3FFF


def _make_sc_agg(num_srcs):
    mesh = plsc.VectorSubcoreMesh(core_axis_name="c", subcore_axis_name="s")

    @functools.partial(
        pl.kernel,
        mesh=mesh,
        out_type=jax.ShapeDtypeStruct((C, num_srcs, N, D), jnp.float32),
        scratch_types=[
            pltpu.VMEM((E_PAD,), jnp.int32),
            pltpu.VMEM((1, CHUNK), jnp.int32),
            pltpu.VMEM((1, CHUNK), jnp.int32),
            pltpu.VMEM((CHUNK, D), jnp.float32),
            pltpu.VMEM((1, CHUNK), jnp.int32),
            pltpu.VMEM((1, CHUNK), jnp.int32),
            pltpu.VMEM((CHUNK, D), jnp.float32),
            pltpu.VMEM_SHARED((TROWS, D), jnp.float32),
            pltpu.SemaphoreType.DMA,
            pltpu.SemaphoreType.DMA,
            pltpu.SemaphoreType.DMA,
        ],
    )
    def agg_kernel(*refs):
        feats = refs[:num_srcs]
        pk_hbm, zeros_hbm, out_hbm = refs[num_srcs:num_srcs + 3]
        (pk_v, gidx_a, sidx_a, rows_a, gidx_b, sidx_b, rows_b,
         table, sem_a, sem_b, sem_s) = refs[num_srcs + 3:]

        cid = lax.axis_index("c")
        sid = lax.axis_index("s")

        eb = sid * E_TILE
        pltpu.sync_copy(pk_hbm.at[pl.ds(eb, E_TILE)], pk_v.at[pl.ds(0, E_TILE)])
        for k in range((E_PAD - E_TILE) // 16):
            pk_v[pl.ds(E_TILE + k * 16, 16)] = jnp.full((16,), DUMP << 14,
                                                        jnp.int32)

        for r in range(2):
            q = cid * 2 + r

            def unpack(off, gidx, sidx):
                for j in range(NGRP):
                    p16 = pk_v[pl.ds(off + j * 16, 16)]
                    gidx[0, pl.ds(j * 16, 16)] = p16 & MASK14
                    sidx[0, pl.ds(j * 16, 16)] = jnp.where(
                        (p16 >> 28) == q, (p16 >> 14) & MASK14,
                        DUMP + sid * 8 + (j % 8))

            for f in range(num_srcs):
                feat = feats[f]

                zb = sid * ZSTRIPE
                pltpu.sync_copy(zeros_hbm.at[pl.ds(zb, ZSTRIPE)],
                                table.at[pl.ds(zb, ZSTRIPE)])
                plsc.subcore_barrier()

                def pair_steps(k, first):
                    off_a = (2 * k) * CHUNK
                    off_b = off_a + CHUNK
                    if not first:
                        pltpu.make_async_copy(
                            rows_a, table.at[sidx_a.at[0]], sem_s).wait()
                        pltpu.make_async_copy(
                            rows_b, table.at[sidx_b.at[0]], sem_s).wait()
                    unpack(off_a, gidx_a, sidx_a)
                    ga = pltpu.async_copy(feat.at[gidx_a.at[0]], rows_a,
                                          sem_a)
                    unpack(off_b, gidx_b, sidx_b)
                    gb = pltpu.async_copy(feat.at[gidx_b.at[0]], rows_b,
                                          sem_b)
                    ga.wait()
                    pltpu.async_copy(rows_a, table.at[sidx_a.at[0]], sem_s,
                                     add=True)
                    gb.wait()
                    pltpu.async_copy(rows_b, table.at[sidx_b.at[0]], sem_s,
                                     add=True)

                def pair_body(k, carry):
                    pair_steps(k, first=False)
                    return carry

                pair_steps(0, first=True)
                lax.fori_loop(1, NPAIR, pair_body, 0)
                pltpu.make_async_copy(rows_a, table.at[sidx_a.at[0]],
                                      sem_s).wait()
                pltpu.make_async_copy(rows_b, table.at[sidx_b.at[0]],
                                      sem_s).wait()
                unpack((NCHUNK - 1) * CHUNK, gidx_a, sidx_a)
                pltpu.async_copy(feat.at[gidx_a.at[0]], rows_a, sem_a).wait()
                pltpu.async_copy(rows_a, table.at[sidx_a.at[0]], sem_s,
                                 add=True).wait()
                plsc.subcore_barrier()

                db = sid * DSTRIPE
                for qq in range(C):
                    @pl.when(q == qq)
                    def _():
                        pltpu.sync_copy(
                            table.at[pl.ds(db, DSTRIPE)],
                            out_hbm.at[qq, f, pl.ds(db, DSTRIPE)])

                    @pl.when(jnp.logical_and(q == qq, sid == 0))
                    def _():
                        pltpu.sync_copy(
                            table.at[pl.ds(NTILES * DSTRIPE, DREM)],
                            out_hbm.at[qq, f, pl.ds(NTILES * DSTRIPE, DREM)])
                plsc.subcore_barrier()

    return agg_kernel


_sc_agg1 = _make_sc_agg(1)
_sc_agg2 = _make_sc_agg(2)


BN = 1000


def _tc1_body(x_ref, agg_ref, wl_ref, wc_ref, b_ref, ha_ref, hb_ref):
    acc = jnp.dot(x_ref[...], wl_ref[...], preferred_element_type=jnp.float32)
    for c in range(C):
        acc += jnp.dot(agg_ref[c, 0], wc_ref[c],
                       preferred_element_type=jnp.float32)
    acc += b_ref[...]
    h = jnp.maximum(acc, 0.0)
    ha_ref[...] = h[:, :D]
    hb_ref[...] = h[:, D:]


def _tc_layer1(x, agg1, wl1t, wc1t, b1):
    return pl.pallas_call(
        _tc1_body,
        grid=(N // BN,),
        in_specs=[
            pl.BlockSpec((BN, D), lambda i: (i, 0)),
            pl.BlockSpec((C, 1, BN, D), lambda i: (0, 0, i, 0)),
            pl.BlockSpec((D, H), lambda i: (0, 0)),
            pl.BlockSpec((C, D, H), lambda i: (0, 0, 0)),
            pl.BlockSpec((1, H), lambda i: (0, 0)),
        ],
        out_specs=[
            pl.BlockSpec((BN, D), lambda i: (i, 0)),
            pl.BlockSpec((BN, D), lambda i: (i, 0)),
        ],
        out_shape=[
            jax.ShapeDtypeStruct((N, D), jnp.float32),
            jax.ShapeDtypeStruct((N, D), jnp.float32),
        ],
    )(x, agg1, wl1t, wc1t, b1)


def _tc2_body(ha_ref, hb_ref, agg_ref, wl_ref, wc_ref, b_ref, o_ref):
    acc = jnp.dot(ha_ref[...], wl_ref[:D], preferred_element_type=jnp.float32)
    acc += jnp.dot(hb_ref[...], wl_ref[D:], preferred_element_type=jnp.float32)
    for c in range(C):
        for f in range(2):
            acc += jnp.dot(agg_ref[c, f], wc_ref[c, f * D:(f + 1) * D],
                           preferred_element_type=jnp.float32)
    acc += b_ref[...]
    o_ref[...] = jax.nn.sigmoid(acc - 10.0)


def _tc_layer2(ha, hb, agg2, wl2t, wc2t, b2):
    return pl.pallas_call(
        _tc2_body,
        grid=(N // BN,),
        in_specs=[
            pl.BlockSpec((BN, D), lambda i: (i, 0)),
            pl.BlockSpec((BN, D), lambda i: (i, 0)),
            pl.BlockSpec((C, 2, BN, D), lambda i: (0, 0, i, 0)),
            pl.BlockSpec((H, D), lambda i: (0, 0)),
            pl.BlockSpec((C, H, D), lambda i: (0, 0, 0)),
            pl.BlockSpec((1, D), lambda i: (0, 0)),
        ],
        out_specs=pl.BlockSpec((BN, D), lambda i: (i, 0)),
        out_shape=jax.ShapeDtypeStruct((N, D), jnp.float32),
    )(ha, hb, agg2, wl2t, wc2t, b2)


def kernel(x, edge_index, edge_colour, Wc1, Wl1, b1, Wc2, Wl2, b2):
    packed = (edge_index[0] | (edge_index[1] << 14)
              | (edge_colour << 28)).astype(jnp.int32)
    zeros = jnp.zeros((TROWS, D), jnp.float32)

    agg1 = _sc_agg1(x, packed, zeros)
    ha, hb = _tc_layer1(x, agg1, Wl1.T,
                        jnp.transpose(Wc1, (0, 2, 1)), b1.reshape(1, H))
    agg2 = _sc_agg2(ha, hb, packed, zeros)
    out = _tc_layer2(ha, hb, agg2, Wl2.T,
                     jnp.transpose(Wc2, (0, 2, 1)), b2.reshape(1, D))
    return out
